# baseline (speedup 1.0000x reference)
import functools

import jax
import jax.numpy as jnp
from jax.experimental import pallas as pl
from jax.experimental.pallas import tpu as pltpu


def _moe_dense_body(xb_ref, gwT_ref, g_ref, u_ref, d_ref, out_ref,
                    cw_ref, *, topk):
    e = pl.program_id(1)

    @pl.when(e == 0)
    def _gate():
        logits = jax.lax.dot_general(
            xb_ref[...], gwT_ref[...], (((1,), (0,)), ((), ())),
            preferred_element_type=jnp.float32)
        mx = jnp.max(logits, axis=1, keepdims=True)
        ex = jnp.exp(logits - mx)
        p = ex / jnp.sum(ex, axis=1, keepdims=True)
        tm, ne = p.shape
        a = p[:, :, None]
        b = p[:, None, :]
        ii = jax.lax.broadcasted_iota(jnp.int32, (tm, ne, ne), 1)
        jj = jax.lax.broadcasted_iota(jnp.int32, (tm, ne, ne), 2)
        beats = (b > a) | ((b == a) & (jj < ii))
        rank = jnp.sum(beats.astype(jnp.float32), axis=2)
        cw = jnp.where(rank < float(topk), p, 0.0)
        ncols = cw_ref.shape[1]
        lane = jax.lax.broadcasted_iota(jnp.int32, (tm, ncols), 1)
        cw_ref[...] = jnp.where(
            lane < ne,
            jnp.pad(cw, ((0, 0), (0, ncols - ne))),
            1.0)

    lane = jax.lax.broadcasted_iota(jnp.int32, cw_ref.shape, 1)
    w = jnp.sum(jnp.where(lane == e, cw_ref[...], 0.0), axis=1)

    xb = xb_ref[...]
    gw = g_ref[0]
    uw = u_ref[0]
    dw = d_ref[0]
    g = jax.lax.dot_general(xb, gw, (((1,), (1,)), ((), ())),
                            preferred_element_type=jnp.float32)
    u = jax.lax.dot_general(xb, uw, (((1,), (1,)), ((), ())),
                            preferred_element_type=jnp.float32)
    h = (g * jax.nn.sigmoid(g)) * u
    hd = jax.lax.dot_general(h.astype(jnp.bfloat16), dw,
                             (((1,), (1,)), ((), ())),
                             preferred_element_type=jnp.float32)
    contrib = hd * w[:, None]

    @pl.when(e == 0)
    def _init():
        out_ref[...] = contrib

    @pl.when(e > 0)
    def _acc():
        out_ref[...] += contrib


def kernel(hidden_states, gate_w, eg, eu, ed, sg, su, sd):
    orig_shape = hidden_states.shape
    d = orig_shape[-1]
    x = hidden_states.reshape(-1, d)
    n = x.shape[0]
    e_num, dff = eg.shape[0], eg.shape[1]
    nsh = sg.shape[0] // dff
    ne = e_num + nsh

    xb = x.astype(jnp.bfloat16)
    gs = jnp.concatenate([eg, sg.reshape(nsh, dff, d)], 0).astype(jnp.bfloat16)
    us = jnp.concatenate([eu, su.reshape(nsh, dff, d)], 0).astype(jnp.bfloat16)
    ds = jnp.concatenate(
        [ed, sd.reshape(d, nsh, dff).transpose(1, 0, 2)], 0
    ).astype(jnp.bfloat16)
    gwt = gate_w.T.astype(jnp.bfloat16)

    tm = min(512, n)
    n_m = n // tm

    out = pl.pallas_call(
        functools.partial(_moe_dense_body, topk=2),
        grid=(n_m, ne),
        in_specs=[
            pl.BlockSpec((tm, d), lambda m, e: (m, 0)),
            pl.BlockSpec((d, e_num), lambda m, e: (0, 0)),
            pl.BlockSpec((1, dff, d), lambda m, e: (e, 0, 0)),
            pl.BlockSpec((1, dff, d), lambda m, e: (e, 0, 0)),
            pl.BlockSpec((1, d, dff), lambda m, e: (e, 0, 0)),
        ],
        out_specs=pl.BlockSpec((tm, d), lambda m, e: (m, 0)),
        out_shape=jax.ShapeDtypeStruct((n, d), jnp.float32),
        scratch_shapes=[pltpu.VMEM((tm, 16), jnp.float32)],
        compiler_params=pltpu.CompilerParams(
            dimension_semantics=("arbitrary", "arbitrary")),
    )(xb, gwt, gs, us, ds)
    return out.reshape(orig_shape)
